# Initial kernel scaffold; baseline (speedup 1.0000x reference)
import jax
import jax.numpy as jnp
from jax.experimental import pallas as pl


def kernel(x, pos, params, edge_index, batch):
    raise NotImplementedError("write your pallas kernel here")


# speedup vs baseline: 2.7456x; 2.7456x over previous
import functools

import jax
import jax.numpy as jnp
from jax import lax
from jax.experimental import pallas as pl
from jax.experimental.pallas import tpu as pltpu
from jax.experimental.pallas import tpu_sc as plsc

N = 10000
E = 320000
H = 128
G = 16
TW = 144
CW = 50
NROW = E // CW
NWORK = 32
RPW = NROW // NWORK
CHUNK = 8
NCHUNK = RPW // CHUNK
SCH = 4
NSCH = RPW // SCH
NSUB = 16
NP = 10240
RSUB = NP // NSUB
BN = 2000
BE = 1000


def _ln(v, g, b):
    mu = jnp.mean(v, axis=-1, keepdims=True)
    var = jnp.mean((v - mu) ** 2, axis=-1, keepdims=True)
    return (v - mu) / jnp.sqrt(var + 1e-5) * g + b


def _dot(a, b):
    return jnp.dot(a, b, preferred_element_type=jnp.float32)


def _sc_gather(T, src3, dst3):
    mesh = plsc.VectorSubcoreMesh(core_axis_name="c", subcore_axis_name="s")

    @functools.partial(
        pl.kernel,
        out_type=(
            jax.ShapeDtypeStruct((NROW, CW, TW), jnp.float32),
            jax.ShapeDtypeStruct((NROW, CW, TW), jnp.float32),
        ),
        mesh=mesh,
        scratch_types=[
            pltpu.VMEM((CHUNK, 1, CW), jnp.int32),
            pltpu.VMEM((CHUNK, 1, CW), jnp.int32),
            pltpu.VMEM((CHUNK, CW, TW), jnp.float32),
            pltpu.VMEM((CHUNK, CW, TW), jnp.float32),
            pltpu.SemaphoreType.DMA,
        ],
        compiler_params=pltpu.CompilerParams(use_tc_tiling_on_sc=False),
    )
    def k(t_hbm, src_hbm, dst_hbm, gd_hbm, gs_hbm, idxd, idxs, bufd, bufs, sem):
        c = lax.axis_index("c")
        s = lax.axis_index("s")
        w = c * NSUB + s

        @pl.loop(0, NCHUNK)
        def _(t):
            r0 = (w + NWORK * t) * CHUNK
            pltpu.sync_copy(dst_hbm.at[pl.ds(r0, CHUNK)], idxd)
            pltpu.sync_copy(src_hbm.at[pl.ds(r0, CHUNK)], idxs)
            descs = []
            for j in range(CHUNK):
                descs.append(pltpu.async_copy(
                    t_hbm.at[idxd.at[j, 0]], bufd.at[j], sem))
                descs.append(pltpu.async_copy(
                    t_hbm.at[idxs.at[j, 0]], bufs.at[j], sem))
            for d in descs:
                d.wait()
            pltpu.sync_copy(bufd, gd_hbm.at[pl.ds(r0, CHUNK)])
            pltpu.sync_copy(bufs, gs_hbm.at[pl.ds(r0, CHUNK)])

    gd, gs = k(T, src3, dst3)
    return gd.reshape(E, TW), gs.reshape(E, TW)


def _sc_scatter(m, sm, dst3, z128, z8):
    mesh = plsc.VectorSubcoreMesh(core_axis_name="c", subcore_axis_name="s")
    m3 = m.reshape(NROW, CW, H)
    sm3 = sm.reshape(NROW, CW, 8)

    @functools.partial(
        pl.kernel,
        out_type=(
            jax.ShapeDtypeStruct((2, NP, H), jnp.float32),
            jax.ShapeDtypeStruct((2, NP, 8), jnp.float32),
        ),
        mesh=mesh,
        scratch_types=[
            pltpu.VMEM_SHARED((NP, H), jnp.float32),
            pltpu.VMEM_SHARED((NP, 8), jnp.float32),
            pltpu.VMEM((SCH, 1, CW), jnp.int32),
            pltpu.VMEM((SCH, CW, H), jnp.float32),
            pltpu.VMEM((SCH, CW, 8), jnp.float32),
        ],
        compiler_params=pltpu.CompilerParams(use_tc_tiling_on_sc=False),
    )
    def k(m_hbm, sm_hbm, dst_hbm, z128_hbm, z8_hbm, am_hbm, as_hbm,
          am_acc, as_acc, idx, mbuf, smbuf):
        c = lax.axis_index("c")
        s = lax.axis_index("s")
        w = c * NSUB + s

        pltpu.sync_copy(z128_hbm, am_acc.at[pl.ds(s * RSUB, RSUB)])
        pltpu.sync_copy(z8_hbm, as_acc.at[pl.ds(s * RSUB, RSUB)])
        plsc.subcore_barrier()

        @pl.loop(0, NSCH)
        def _(t):
            r0 = (w + NWORK * t) * SCH
            pltpu.sync_copy(dst_hbm.at[pl.ds(r0, SCH)], idx)
            pltpu.sync_copy(m_hbm.at[pl.ds(r0, SCH)], mbuf)
            pltpu.sync_copy(sm_hbm.at[pl.ds(r0, SCH)], smbuf)
            for j in range(SCH):
                pltpu.sync_copy(mbuf.at[j], am_acc.at[idx.at[j, 0]], add=True)
                pltpu.sync_copy(smbuf.at[j], as_acc.at[idx.at[j, 0]], add=True)

        plsc.subcore_barrier()
        pltpu.sync_copy(am_acc.at[pl.ds(s * RSUB, RSUB)],
                        am_hbm.at[c, pl.ds(s * RSUB, RSUB)])
        pltpu.sync_copy(as_acc.at[pl.ds(s * RSUB, RSUB)],
                        as_hbm.at[c, pl.ds(s * RSUB, RSUB)])

    return k(m3, sm3, dst3, z128, z8)


def _tc_embed(x, pos4, w, b):
    def body(x_ref, p_ref, w_ref, b_ref, t_ref):
        h = _dot(x_ref[...], w_ref[...]) + b_ref[...]
        t_ref[...] = jnp.concatenate(
            [h, p_ref[...], jnp.zeros((BN, TW - H - 4), jnp.float32)], axis=1)

    return pl.pallas_call(
        body,
        grid=(N // BN,),
        in_specs=[
            pl.BlockSpec((BN, H), lambda i: (i, 0)),
            pl.BlockSpec((BN, 4), lambda i: (i, 0)),
            pl.BlockSpec((H, H), lambda i: (0, 0)),
            pl.BlockSpec((1, H), lambda i: (0, 0)),
        ],
        out_specs=pl.BlockSpec((BN, TW), lambda i: (i, 0)),
        out_shape=jax.ShapeDtypeStruct((N, TW), jnp.float32),
    )(x, pos4, w, b)


def _tc_edge(gd, gs, wts):
    (w1a, w1b, w1c, b1, g1, be1, w2, b2, g2, be2,
     pw1, pb1, pg1, pbe1, pw2, pb2) = wts

    def body(gd_ref, gs_ref, w1a_r, w1b_r, w1c_r, b1_r, g1_r, be1_r,
             w2_r, b2_r, g2_r, be2_r, pw1_r, pb1_r, pg1_r, pbe1_r,
             pw2_r, pb2_r, m_ref, sm_ref):
        gdv = gd_ref[...]
        gsv = gs_ref[...]
        hd = gdv[:, :H]
        hs = gsv[:, :H]
        pdiff = gdv[:, H:H + 4] - gsv[:, H:H + 4]
        dist = jnp.sqrt(jnp.sum(pdiff * pdiff, axis=-1, keepdims=True) + 1e-12)
        z = _dot(hd, w1a_r[...]) + _dot(hs, w1b_r[...]) + dist * w1c_r[...] + b1_r[...]
        m1 = jax.nn.relu(_ln(z, g1_r[...], be1_r[...]))
        m2 = jax.nn.relu(_ln(_dot(m1, w2_r[...]) + b2_r[...], g2_r[...], be2_r[...]))
        pw = jax.nn.relu(_ln(_dot(m2, pw1_r[...]) + pb1_r[...], pg1_r[...], pbe1_r[...]))
        sc = _dot(pw, pw2_r[...]) + pb2_r[...]
        posd = pdiff * sc
        m_ref[...] = m2
        sm_ref[...] = jnp.concatenate(
            [posd, jnp.ones((BE, 1), jnp.float32), jnp.zeros((BE, 3), jnp.float32)],
            axis=1)

    full = lambda shape: pl.BlockSpec(shape, lambda i: (0, 0))
    return pl.pallas_call(
        body,
        grid=(E // BE,),
        in_specs=[
            pl.BlockSpec((BE, TW), lambda i: (i, 0)),
            pl.BlockSpec((BE, TW), lambda i: (i, 0)),
            full((H, H)), full((H, H)), full((1, H)), full((1, H)),
            full((1, H)), full((1, H)),
            full((H, H)), full((1, H)), full((1, H)), full((1, H)),
            full((H, H)), full((1, H)), full((1, H)), full((1, H)),
            full((H, 1)), full((1, 1)),
        ],
        out_specs=[
            pl.BlockSpec((BE, H), lambda i: (i, 0)),
            pl.BlockSpec((BE, 8), lambda i: (i, 0)),
        ],
        out_shape=[
            jax.ShapeDtypeStruct((E, H), jnp.float32),
            jax.ShapeDtypeStruct((E, 8), jnp.float32),
        ],
    )(gd, gs, w1a, w1b, w1c, b1, g1, be1, w2, b2, g2, be2,
      pw1, pb1, pg1, pbe1, pw2, pb2)


def _tc_update(T, am, asml, wts):
    (u1a, u1b, ub1, ug1, ube1, u2, ub2, ug2, ube2) = wts

    def body(t_ref, am0_r, am1_r, as0_r, as1_r, u1a_r, u1b_r, ub1_r, ug1_r,
             ube1_r, u2_r, ub2_r, ug2_r, ube2_r, tn_ref):
        tv = t_ref[...]
        h = tv[:, :H]
        pos4 = tv[:, H:H + 4]
        msg = am0_r[0] + am1_r[0]
        a8 = as0_r[0] + as1_r[0]
        cnt = a8[:, 4:5]
        posadd = a8[:, :4] / jnp.maximum(cnt, 1.0)
        z = _dot(h, u1a_r[...]) + _dot(msg, u1b_r[...]) + ub1_r[...]
        u = jax.nn.relu(_ln(z, ug1_r[...], ube1_r[...]))
        u = jax.nn.relu(_ln(_dot(u, u2_r[...]) + ub2_r[...], ug2_r[...], ube2_r[...]))
        hn = h + u
        posn = pos4 + posadd
        tn_ref[...] = jnp.concatenate(
            [hn, posn, jnp.zeros((BN, TW - H - 4), jnp.float32)], axis=1)

    full = lambda shape: pl.BlockSpec(shape, lambda i: (0, 0))
    nb = N // BN
    return pl.pallas_call(
        body,
        grid=(nb,),
        in_specs=[
            pl.BlockSpec((BN, TW), lambda i: (i, 0)),
            pl.BlockSpec((1, BN, H), lambda i: (0, i, 0)),
            pl.BlockSpec((1, BN, H), lambda i: (1, i, 0)),
            pl.BlockSpec((1, BN, 8), lambda i: (0, i, 0)),
            pl.BlockSpec((1, BN, 8), lambda i: (1, i, 0)),
            full((H, H)), full((H, H)), full((1, H)), full((1, H)), full((1, H)),
            full((H, H)), full((1, H)), full((1, H)), full((1, H)),
        ],
        out_specs=pl.BlockSpec((BN, TW), lambda i: (i, 0)),
        out_shape=jax.ShapeDtypeStruct((N, TW), jnp.float32),
    )(T, am, am, asml, asml, u1a, u1b, ub1, ug1, ube1, u2, ub2, ug2, ube2)


def _tc_readout(T, batch2, pw1, pb1, pw2, pb2):
    def body(t_ref, b_ref, pw1_r, pb1_r, pw2_r, pb2_r, o_ref):
        h = t_ref[...][:, :H]
        oh = (lax.broadcasted_iota(jnp.int32, (G, N), 0) == b_ref[...]
              ).astype(jnp.float32)
        gsum = _dot(oh, h)
        r = jax.nn.relu(_dot(gsum, pw1_r[...]) + pb1_r[...])
        o_ref[...] = _dot(r, pw2_r[...]) + pb2_r[...]

    full = lambda shape: pl.BlockSpec(shape, lambda i: (0, 0))
    return pl.pallas_call(
        body,
        grid=(1,),
        in_specs=[
            full((N, TW)), full((1, N)),
            full((H, H)), full((1, H)), full((H, 1)), full((1, 1)),
        ],
        out_specs=full((G, 1)),
        out_shape=jax.ShapeDtypeStruct((G, 1), jnp.float32),
    )(T, batch2, pw1, pb1, pw2, pb2)


def _r2(v):
    return v.reshape(1, -1)


def kernel(x, pos, params, edge_index, batch):
    src3 = edge_index[0].reshape(NROW, 1, CW)
    dst3 = edge_index[1].reshape(NROW, 1, CW)
    pos4 = jnp.pad(pos, ((0, 0), (0, 1)))
    batch2 = batch.reshape(1, N)
    z128 = jnp.zeros((RSUB, H), jnp.float32)
    z8 = jnp.zeros((RSUB, 8), jnp.float32)

    T = _tc_embed(x, pos4, params['emb_W'], _r2(params['emb_b']))
    for lp in params['layers']:
        gd, gs = _sc_gather(T, src3, dst3)
        ewts = (lp['msg_W1'][:H], lp['msg_W1'][H:2 * H], lp['msg_W1'][2 * H:],
                _r2(lp['msg_b1']), _r2(lp['msg_g1']), _r2(lp['msg_be1']),
                lp['msg_W2'], _r2(lp['msg_b2']), _r2(lp['msg_g2']),
                _r2(lp['msg_be2']),
                lp['pos_W1'], _r2(lp['pos_b1']), _r2(lp['pos_g1']),
                _r2(lp['pos_be1']), lp['pos_W2'], _r2(lp['pos_b2']))
        m, sm = _tc_edge(gd, gs, ewts)
        am, asml = _sc_scatter(m, sm, dst3, z128, z8)
        uwts = (lp['upd_W1'][:H], lp['upd_W1'][H:], _r2(lp['upd_b1']),
                _r2(lp['upd_g1']), _r2(lp['upd_be1']),
                lp['upd_W2'], _r2(lp['upd_b2']), _r2(lp['upd_g2']),
                _r2(lp['upd_be2']))
        T = _tc_update(T, am, asml, uwts)

    return _tc_readout(T, batch2, params['pred_W1'], _r2(params['pred_b1']),
                       params['pred_W2'], _r2(params['pred_b2']))
